# Initial kernel scaffold; baseline (speedup 1.0000x reference)
import jax
import jax.numpy as jnp
from jax.experimental import pallas as pl


def kernel(scalar_short, scalar_long, vector_short, vector_long, fragment_ids, W1, b1, ln_g, ln_b, W2, b2, Wv, Wt1, bt1, Wt2, bt2):
    raise NotImplementedError("write your pallas kernel here")


# speedup vs baseline: 12.4568x; 12.4568x over previous
import jax
import jax.numpy as jnp
from jax import lax
from jax.experimental import pallas as pl
from jax.experimental.pallas import tpu as pltpu

H = 128
NV = 16
F = 1000
FPAD = 1152
FOUT = 1024
B = 3200
W = 128
XC = H + 4
MIN_FRAG = 2.0
TEMP = 1.0
SLR = 0.5
VW = 0.1
CF = 0.05
DELTA = 0.1
NPAIR = F * (F - 1) // 2

HI = lax.Precision.HIGHEST


def _silu(x):
    return x * jax.nn.sigmoid(x)


def _body(ids_ref, ss_ref, sl_ref, vs_ref, vl_ref,
          W1_ref, b1_ref, g_ref, bln_ref, W2_ref, b2_ref, Wv_ref,
          Wt1_ref, bt1_ref, Wt2_ref, bt2_ref,
          out_ref, acc_ref):
    i = pl.program_id(0)
    nsteps = pl.num_programs(0)

    @pl.when(i == 0)
    def _init():
        acc_ref[...] = jnp.zeros_like(acc_ref)

    sc = ss_ref[...] * SLR + sl_ref[...] * (1.0 - SLR)
    z = jnp.dot(sc, W1_ref[...], preferred_element_type=jnp.float32,
                precision=HI) + b1_ref[0, :]
    mu = jnp.mean(z, axis=-1, keepdims=True)
    var = jnp.mean((z - mu) ** 2, axis=-1, keepdims=True)
    zn = (z - mu) / jnp.sqrt(var + 1e-5) * g_ref[0, :] + bln_ref[0, :]
    h = _silu(zn)
    sp = jnp.dot(h, W2_ref[...], preferred_element_type=jnp.float32,
                 precision=HI) + b2_ref[0, :]
    sq = jnp.sum(sp * sp, axis=1, keepdims=True)

    q = vs_ref[...] * SLR + vl_ref[...] * (1.0 - SLR)
    q2 = q * q
    sel = (lax.broadcasted_iota(jnp.int32, (3 * NV, NV), 0) // 3 ==
           lax.broadcasted_iota(jnp.int32, (3 * NV, NV), 1)).astype(jnp.float32)
    vn = jnp.sqrt(jnp.dot(q2, sel, preferred_element_type=jnp.float32,
                          precision=HI))
    vmf = jnp.sum(vn, axis=1, keepdims=True) * (1.0 / NV)

    x = jnp.concatenate(
        [sp, sq, vmf, vmf * vmf, jnp.ones_like(vmf)], axis=1)

    idc = ids_ref[0]
    m8 = (jnp.min(idc) // 8) * 8
    nwin = (jnp.max(idc) - m8) // W + 1

    def win(w, _):
        ws = m8 + w * W
        lane = lax.broadcasted_iota(jnp.int32, (B, W), 1) + ws
        oh = (idc == lane).astype(jnp.float32)
        part = lax.dot_general(oh, x, (((0,), (0,)), ((), ())),
                               preferred_element_type=jnp.float32,
                               precision=HI)
        acc_ref[pl.ds(ws, W), :] += part
        return 0

    lax.fori_loop(0, nwin, win, 0)

    @pl.when(i == nsteps - 1)
    def _fin():
        acc = acc_ref[:FOUT, :]
        s1 = acc[:, :H]
        s2 = acc[:, H:H + 1]
        sv1 = acc[:, H + 1:H + 2]
        sv2 = acc[:, H + 2:H + 3]
        cnt = acc[:, H + 3:H + 4]
        cnt1 = jnp.maximum(cnt, 1.0)

        fmask = lax.broadcasted_iota(jnp.int32, (FOUT, 1), 0) < F
        valid = (cnt >= MIN_FRAG) & fmask
        vc = jnp.sum(valid.astype(jnp.float32))
        inv_vc = 1.0 / jnp.maximum(vc, 1.0)

        gf = s1 / cnt1
        m2 = jnp.sum(s1 * s1, axis=1, keepdims=True)
        varf = jnp.maximum((s2 - m2 / cnt1) / cnt1, 0.0)
        sqv = jnp.sqrt(varf + 1e-8)
        intra = jnp.where(vc > 0.0,
                          jnp.sum(jnp.where(valid, sqv, 0.0)) * inv_vc, 0.0)

        cw = jnp.sum(Wv_ref[...], axis=0, keepdims=True)
        s_w = jnp.sum(cw * cw)
        varv = jnp.maximum((sv2 - sv1 * sv1 / cnt1) / cnt1, 0.0) * s_w
        sqvv = jnp.sqrt(varv + 1e-8)
        vloss = jnp.where(vc > 0.0,
                          jnp.sum(jnp.where(valid, sqvv, 0.0)) * inv_vc, 0.0)

        u = _silu(jnp.dot(gf, Wt1_ref[...], preferred_element_type=jnp.float32,
                          precision=HI) + bt1_ref[0, :])
        tl = jnp.sum(u * Wt2_ref[0, :][None, :], axis=1,
                     keepdims=True) + bt2_ref[0, 0]
        t = jnp.clip(jax.nn.sigmoid(tl), 0.2, 0.8)
        avg_t = jnp.sum(jnp.where(fmask, t, 0.0)) * (1.0 / F)

        nrm = jnp.sqrt(jnp.sum(gf * gf, axis=1, keepdims=True))
        nf = gf / jnp.maximum(nrm, 1e-12)
        sim = lax.dot_general(nf, nf, (((1,), (1,)), ((), ())),
                              preferred_element_type=jnp.float32,
                              precision=HI) * (1.0 / TEMP)
        diff = sim - avg_t
        ad = jnp.abs(diff)
        hub = jnp.where(ad <= DELTA, 0.5 * diff * diff,
                        DELTA * (ad - 0.5 * DELTA))
        ri = lax.broadcasted_iota(jnp.int32, (FOUT, FOUT), 0)
        rj = lax.broadcasted_iota(jnp.int32, (FOUT, FOUT), 1)
        pmask = (ri < F) & (rj < F) & (ri != rj)
        gsl = jnp.sum(jnp.where(pmask, hub, 0.0)) * (0.5 / NPAIR)

        total = 0.3 * intra + 0.7 * gsl + VW * vloss
        out_ref[...] = jnp.full((1, 1), CF * total, dtype=jnp.float32)


def kernel(scalar_short, scalar_long, vector_short, vector_long, fragment_ids,
           W1, b1, ln_g, ln_b, W2, b2, Wv, Wt1, bt1, Wt2, bt2):
    n = scalar_short.shape[0]
    g = n // B
    ids3 = fragment_ids.astype(jnp.int32).reshape(g, B, 1)
    vs2 = vector_short.reshape(n, 3 * NV)
    vl2 = vector_long.reshape(n, 3 * NV)

    row = lambda v: v.reshape(1, -1)

    grid_spec = pltpu.PrefetchScalarGridSpec(
        num_scalar_prefetch=0,
        grid=(g,),
        in_specs=[
            pl.BlockSpec((1, B, 1), lambda i: (i, 0, 0)),
            pl.BlockSpec((B, H), lambda i: (i, 0)),
            pl.BlockSpec((B, H), lambda i: (i, 0)),
            pl.BlockSpec((B, 3 * NV), lambda i: (i, 0)),
            pl.BlockSpec((B, 3 * NV), lambda i: (i, 0)),
            pl.BlockSpec((H, H), lambda i: (0, 0)),
            pl.BlockSpec((1, H), lambda i: (0, 0)),
            pl.BlockSpec((1, H), lambda i: (0, 0)),
            pl.BlockSpec((1, H), lambda i: (0, 0)),
            pl.BlockSpec((H, H), lambda i: (0, 0)),
            pl.BlockSpec((1, H), lambda i: (0, 0)),
            pl.BlockSpec((H, H), lambda i: (0, 0)),
            pl.BlockSpec((H, 32), lambda i: (0, 0)),
            pl.BlockSpec((1, 32), lambda i: (0, 0)),
            pl.BlockSpec((1, 32), lambda i: (0, 0)),
            pl.BlockSpec((1, 1), lambda i: (0, 0)),
        ],
        out_specs=pl.BlockSpec((1, 1), lambda i: (0, 0)),
        scratch_shapes=[pltpu.VMEM((FPAD, XC), jnp.float32)],
    )

    out = pl.pallas_call(
        _body,
        grid_spec=grid_spec,
        out_shape=jax.ShapeDtypeStruct((1, 1), jnp.float32),
        compiler_params=pltpu.CompilerParams(
            dimension_semantics=("arbitrary",)),
    )(ids3, scalar_short, scalar_long, vs2, vl2,
      W1, row(b1), row(ln_g), row(ln_b), W2, row(b2), Wv,
      Wt1, row(bt1), row(Wt2), row(bt2))
    return out[0, 0]


# speedup vs baseline: 17.3096x; 1.3896x over previous
import jax
import jax.numpy as jnp
from jax import lax
from jax.experimental import pallas as pl
from jax.experimental.pallas import tpu as pltpu

H = 128
NV = 16
F = 1000
FPAD = 1152
FOUT = 1024
B = 6400
W = 128
XC = H + 4
MIN_FRAG = 2.0
TEMP = 1.0
SLR = 0.5
VW = 0.1
CF = 0.05
DELTA = 0.1
NPAIR = F * (F - 1) // 2

HI = lax.Precision.DEFAULT


def _silu(x):
    return x * jax.nn.sigmoid(x)


def _body(ids_ref, ss_ref, sl_ref, vs_ref, vl_ref,
          W1_ref, b1_ref, g_ref, bln_ref, W2_ref, b2_ref, Wv_ref,
          Wt1_ref, bt1_ref, Wt2_ref, bt2_ref,
          out_ref, acc_ref):
    i = pl.program_id(0)
    nsteps = pl.num_programs(0)

    @pl.when(i == 0)
    def _init():
        acc_ref[...] = jnp.zeros_like(acc_ref)

    sc = ss_ref[...] * SLR + sl_ref[...] * (1.0 - SLR)
    z = jnp.dot(sc, W1_ref[...], preferred_element_type=jnp.float32,
                precision=HI) + b1_ref[0, :]
    mu = jnp.mean(z, axis=-1, keepdims=True)
    var = jnp.mean((z - mu) ** 2, axis=-1, keepdims=True)
    zn = (z - mu) / jnp.sqrt(var + 1e-5) * g_ref[0, :] + bln_ref[0, :]
    h = _silu(zn)
    sp = jnp.dot(h, W2_ref[...], preferred_element_type=jnp.float32,
                 precision=HI) + b2_ref[0, :]
    sq = jnp.sum(sp * sp, axis=1, keepdims=True)

    q = vs_ref[...] * SLR + vl_ref[...] * (1.0 - SLR)
    q2 = q * q
    sel = (lax.broadcasted_iota(jnp.int32, (3 * NV, NV), 0) // 3 ==
           lax.broadcasted_iota(jnp.int32, (3 * NV, NV), 1)).astype(jnp.float32)
    vn = jnp.sqrt(jnp.dot(q2, sel, preferred_element_type=jnp.float32,
                          precision=HI))
    vmf = jnp.sum(vn, axis=1, keepdims=True) * (1.0 / NV)

    x = jnp.concatenate(
        [sp, sq, vmf, vmf * vmf, jnp.ones_like(vmf)], axis=1)

    idc = ids_ref[0]
    m8 = (jnp.min(idc) // 8) * 8
    nwin = (jnp.max(idc) - m8) // W + 1

    def win(w, _):
        ws = m8 + w * W
        lane = lax.broadcasted_iota(jnp.int32, (B, W), 1) + ws
        oh = (idc == lane).astype(jnp.float32)
        part = lax.dot_general(oh, x, (((0,), (0,)), ((), ())),
                               preferred_element_type=jnp.float32,
                               precision=HI)
        acc_ref[pl.ds(ws, W), :] += part
        return 0

    lax.fori_loop(0, nwin, win, 0)

    @pl.when(i == nsteps - 1)
    def _fin():
        acc = acc_ref[:FOUT, :]
        s1 = acc[:, :H]
        s2 = acc[:, H:H + 1]
        sv1 = acc[:, H + 1:H + 2]
        sv2 = acc[:, H + 2:H + 3]
        cnt = acc[:, H + 3:H + 4]
        cnt1 = jnp.maximum(cnt, 1.0)

        fmask = lax.broadcasted_iota(jnp.int32, (FOUT, 1), 0) < F
        valid = (cnt >= MIN_FRAG) & fmask
        vc = jnp.sum(valid.astype(jnp.float32))
        inv_vc = 1.0 / jnp.maximum(vc, 1.0)

        gf = s1 / cnt1
        m2 = jnp.sum(s1 * s1, axis=1, keepdims=True)
        varf = jnp.maximum((s2 - m2 / cnt1) / cnt1, 0.0)
        sqv = jnp.sqrt(varf + 1e-8)
        intra = jnp.where(vc > 0.0,
                          jnp.sum(jnp.where(valid, sqv, 0.0)) * inv_vc, 0.0)

        cw = jnp.sum(Wv_ref[...], axis=0, keepdims=True)
        s_w = jnp.sum(cw * cw)
        varv = jnp.maximum((sv2 - sv1 * sv1 / cnt1) / cnt1, 0.0) * s_w
        sqvv = jnp.sqrt(varv + 1e-8)
        vloss = jnp.where(vc > 0.0,
                          jnp.sum(jnp.where(valid, sqvv, 0.0)) * inv_vc, 0.0)

        u = _silu(jnp.dot(gf, Wt1_ref[...], preferred_element_type=jnp.float32,
                          precision=HI) + bt1_ref[0, :])
        tl = jnp.sum(u * Wt2_ref[0, :][None, :], axis=1,
                     keepdims=True) + bt2_ref[0, 0]
        t = jnp.clip(jax.nn.sigmoid(tl), 0.2, 0.8)
        avg_t = jnp.sum(jnp.where(fmask, t, 0.0)) * (1.0 / F)

        nrm = jnp.sqrt(jnp.sum(gf * gf, axis=1, keepdims=True))
        nf = gf / jnp.maximum(nrm, 1e-12)
        sim = lax.dot_general(nf, nf, (((1,), (1,)), ((), ())),
                              preferred_element_type=jnp.float32,
                              precision=HI) * (1.0 / TEMP)
        diff = sim - avg_t
        ad = jnp.abs(diff)
        hub = jnp.where(ad <= DELTA, 0.5 * diff * diff,
                        DELTA * (ad - 0.5 * DELTA))
        ri = lax.broadcasted_iota(jnp.int32, (FOUT, FOUT), 0)
        rj = lax.broadcasted_iota(jnp.int32, (FOUT, FOUT), 1)
        pmask = (ri < F) & (rj < F) & (ri != rj)
        gsl = jnp.sum(jnp.where(pmask, hub, 0.0)) * (0.5 / NPAIR)

        total = 0.3 * intra + 0.7 * gsl + VW * vloss
        out_ref[...] = jnp.full((1, 1), CF * total, dtype=jnp.float32)


def kernel(scalar_short, scalar_long, vector_short, vector_long, fragment_ids,
           W1, b1, ln_g, ln_b, W2, b2, Wv, Wt1, bt1, Wt2, bt2):
    n = scalar_short.shape[0]
    g = n // B
    ids3 = fragment_ids.astype(jnp.int32).reshape(g, B, 1)
    vs2 = vector_short.reshape(n, 3 * NV)
    vl2 = vector_long.reshape(n, 3 * NV)

    row = lambda v: v.reshape(1, -1)

    grid_spec = pltpu.PrefetchScalarGridSpec(
        num_scalar_prefetch=0,
        grid=(g,),
        in_specs=[
            pl.BlockSpec((1, B, 1), lambda i: (i, 0, 0)),
            pl.BlockSpec((B, H), lambda i: (i, 0)),
            pl.BlockSpec((B, H), lambda i: (i, 0)),
            pl.BlockSpec((B, 3 * NV), lambda i: (i, 0)),
            pl.BlockSpec((B, 3 * NV), lambda i: (i, 0)),
            pl.BlockSpec((H, H), lambda i: (0, 0)),
            pl.BlockSpec((1, H), lambda i: (0, 0)),
            pl.BlockSpec((1, H), lambda i: (0, 0)),
            pl.BlockSpec((1, H), lambda i: (0, 0)),
            pl.BlockSpec((H, H), lambda i: (0, 0)),
            pl.BlockSpec((1, H), lambda i: (0, 0)),
            pl.BlockSpec((H, H), lambda i: (0, 0)),
            pl.BlockSpec((H, 32), lambda i: (0, 0)),
            pl.BlockSpec((1, 32), lambda i: (0, 0)),
            pl.BlockSpec((1, 32), lambda i: (0, 0)),
            pl.BlockSpec((1, 1), lambda i: (0, 0)),
        ],
        out_specs=pl.BlockSpec((1, 1), lambda i: (0, 0)),
        scratch_shapes=[pltpu.VMEM((FPAD, XC), jnp.float32)],
    )

    out = pl.pallas_call(
        _body,
        grid_spec=grid_spec,
        out_shape=jax.ShapeDtypeStruct((1, 1), jnp.float32),
        compiler_params=pltpu.CompilerParams(
            dimension_semantics=("arbitrary",)),
    )(ids3, scalar_short, scalar_long, vs2, vl2,
      W1, row(b1), row(ln_g), row(ln_b), W2, row(b2), Wv,
      Wt1, row(bt1), row(Wt2), row(bt2))
    return out[0, 0]
